# baseline (speedup 1.0000x reference)
import jax
import jax.numpy as jnp
import numpy as np
from jax.experimental import pallas as pl
from jax.experimental.pallas import tpu as pltpu

B, L, C, D, E, H, K, KER = 512, 336, 21, 256, 8, 256, 2, 25
BB = 64
RB = BB * C


def _build_fold_matrix() -> np.ndarray:
    pad = (KER - 1) // 2
    M = np.zeros((L, L), np.float64)
    for l in range(L):
        for j in range(l - pad, l + pad + 1):
            M[l, min(max(j, 0), L - 1)] += 1.0
    M /= KER
    Mt = M.T
    return np.concatenate([np.eye(L) - Mt, Mt], axis=1).astype(np.float32)


_G = _build_fold_matrix()
_REP = np.equal.outer(np.arange(RB) // C, np.arange(BB)).astype(np.float32)


def _weff_body(ws_ref, wt_ref, g_ref, weff_ref):
    wcat = jnp.concatenate([ws_ref[0], wt_ref[0]], axis=1)
    weff_ref[0] = jax.lax.dot_general(
        g_ref[...], wcat,
        dimension_numbers=(((1,), (1,)), ((), ())),
        preferred_element_type=jnp.float32)


def _fold_weights(ws_s, ws_t, g):
    return pl.pallas_call(
        _weff_body,
        grid=(E,),
        in_specs=[
            pl.BlockSpec((1, D, L), lambda e: (e, 0, 0)),
            pl.BlockSpec((1, D, L), lambda e: (e, 0, 0)),
            pl.BlockSpec((L, 2 * L), lambda e: (0, 0)),
        ],
        out_specs=pl.BlockSpec((1, L, D), lambda e: (e, 0, 0)),
        out_shape=jax.ShapeDtypeStruct((E, L, D), jnp.float32),
    )(ws_s, ws_t, g)


def _revin_gate_body(x_ref, rw_ref, rb_ref, w1_ref, w2_ref,
                     xn_ref, gates_ref):
    xt = jnp.transpose(x_ref[...], (0, 2, 1))
    mu = jnp.mean(xt, axis=2, keepdims=True)
    var = jnp.mean((xt - mu) ** 2, axis=2, keepdims=True)
    sd = jnp.sqrt(var + 1e-5)
    rw = rw_ref[...].reshape(1, C, 1)
    rb = rb_ref[...].reshape(1, C, 1)
    xn = (xt - mu) / sd * rw + rb
    xn_ref[...] = xn

    m = jnp.mean(xn, axis=1)
    h = jax.nn.relu(jnp.dot(m, w1_ref[...],
                            preferred_element_type=jnp.float32))
    logits = jnp.dot(h, w2_ref[...],
                     preferred_element_type=jnp.float32)
    lmax = jnp.max(logits, axis=1, keepdims=True)
    ex = jnp.exp(logits - lmax)
    p = ex / jnp.sum(ex, axis=1, keepdims=True)
    iota = jax.lax.broadcasted_iota(jnp.int32, p.shape, 1)
    v1 = jnp.max(p, axis=1, keepdims=True)
    i1 = jnp.min(jnp.where(p == v1, iota, E), axis=1, keepdims=True)
    p2 = jnp.where(iota == i1, -jnp.inf, p)
    v2 = jnp.max(p2, axis=1, keepdims=True)
    i2 = jnp.min(jnp.where(p2 == v2, iota, E), axis=1, keepdims=True)
    denom = v1 + v2 + 1e-6
    gates_ref[...] = (jnp.where(iota == i1, v1 / denom, 0.0)
                      + jnp.where(iota == i2, v2 / denom, 0.0))


def _revin_gate(x, rw, rb, w1, w2):
    return pl.pallas_call(
        _revin_gate_body,
        grid=(B // BB,),
        in_specs=[
            pl.BlockSpec((BB, L, C), lambda i: (i, 0, 0)),
            pl.BlockSpec((C,), lambda i: (0,)),
            pl.BlockSpec((C,), lambda i: (0,)),
            pl.BlockSpec((L, H), lambda i: (0, 0)),
            pl.BlockSpec((H, E), lambda i: (0, 0)),
        ],
        out_specs=[
            pl.BlockSpec((BB, C, L), lambda i: (i, 0, 0)),
            pl.BlockSpec((BB, E), lambda i: (i, 0)),
        ],
        out_shape=[
            jax.ShapeDtypeStruct((B, C, L), jnp.float32),
            jax.ShapeDtypeStruct((B, E), jnp.float32),
        ],
    )(x, rw, rb, w1, w2)


def _loss_body(gates_ref, out_ref):
    g = gates_ref[...]
    importance = jnp.sum(g, axis=0)
    load = jnp.sum((g > 0).astype(jnp.float32), axis=0)

    def cv2(v):
        mean = jnp.mean(v)
        var = jnp.sum((v - mean) ** 2) / (E - 1)
        return var / (mean ** 2 + 1e-10)

    out_ref[0, 0] = cv2(importance) + cv2(load)


def _loss(gates):
    return pl.pallas_call(
        _loss_body,
        out_shape=jax.ShapeDtypeStruct((1, 1), jnp.float32),
        out_specs=pl.BlockSpec(memory_space=pltpu.SMEM),
    )(gates)


def _moe_body(xn_ref, gates_ref, weff_ref, bias_ref, rep_ref, y_ref):
    xr = xn_ref[...]
    grows = jnp.dot(rep_ref[...], gates_ref[...],
                    preferred_element_type=jnp.float32)
    acc = jnp.dot(grows, bias_ref[...],
                  preferred_element_type=jnp.float32)
    for e in range(E):
        pe = jnp.dot(xr, weff_ref[e],
                     preferred_element_type=jnp.float32)
        acc = acc + grows[:, e:e + 1] * pe
    y_ref[...] = acc


def _moe_matmul(xn_rows, gates, weff, bias, rep):
    return pl.pallas_call(
        _moe_body,
        grid=((B * C) // RB,),
        in_specs=[
            pl.BlockSpec((RB, L), lambda i: (i, 0)),
            pl.BlockSpec((BB, E), lambda i: (i, 0)),
            pl.BlockSpec((E, L, D), lambda i: (0, 0, 0)),
            pl.BlockSpec((E, D), lambda i: (0, 0)),
            pl.BlockSpec((RB, BB), lambda i: (0, 0)),
        ],
        out_specs=pl.BlockSpec((RB, D), lambda i: (i, 0)),
        out_shape=jax.ShapeDtypeStruct((B * C, D), jnp.float32),
    )(xn_rows, gates, weff, bias, rep)


def kernel(x, revin_w, revin_b, gate_W1, gate_W2, Ws_s, bs_s, Ws_t, bs_t):
    g = jnp.asarray(_G)
    rep = jnp.asarray(_REP)
    weff = _fold_weights(Ws_s, Ws_t, g)
    xn_t, gates = _revin_gate(x, revin_w, revin_b, gate_W1, gate_W2)
    loss = _loss(gates)[0, 0]
    bias = bs_s + bs_t
    xn_rows = xn_t.reshape(B * C, L)
    y = _moe_matmul(xn_rows, gates, weff, bias, rep)
    return y.reshape(B, C, D), loss


# speedup vs baseline: 1.2724x; 1.2724x over previous
import jax
import jax.numpy as jnp
import numpy as np
from jax.experimental import pallas as pl
from jax.experimental.pallas import tpu as pltpu

B, L, C, D, E, H, K, KER = 512, 336, 21, 256, 8, 256, 2, 25
BB = 64
NB = B // BB
RB = BB * C


def _build_fold_matrix() -> np.ndarray:
    pad = (KER - 1) // 2
    M = np.zeros((L, L), np.float64)
    for l in range(L):
        for j in range(l - pad, l + pad + 1):
            M[l, min(max(j, 0), L - 1)] += 1.0
    M /= KER
    Mt = M.T
    return np.concatenate([np.eye(L) - Mt, Mt], axis=1).astype(np.float32)


_G = _build_fold_matrix()
_REP = np.equal.outer(np.arange(RB) // C, np.arange(BB)).astype(np.float32)


def _body(x_ref, rw_ref, rb_ref, w1_ref, w2_ref, wss_ref, bss_ref,
          wst_ref, bst_ref, g_ref, rep_ref,
          y_ref, gates_ref, loss_ref, weff_ref):
    i = pl.program_id(0)

    @pl.when(i == 0)
    def _():
        for e in range(E):
            wcat = jnp.concatenate([wss_ref[e], wst_ref[e]], axis=1)
            weff_ref[e] = jax.lax.dot_general(
                g_ref[...], wcat,
                dimension_numbers=(((1,), (1,)), ((), ())),
                preferred_element_type=jnp.float32)

    xt = jnp.transpose(x_ref[...], (0, 2, 1))
    mu = jnp.mean(xt, axis=2, keepdims=True)
    var = jnp.mean((xt - mu) ** 2, axis=2, keepdims=True)
    sd = jnp.sqrt(var + 1e-5)
    rw = rw_ref[...].reshape(1, C, 1)
    rb = rb_ref[...].reshape(1, C, 1)
    xn = (xt - mu) / sd * rw + rb

    m = jnp.mean(xn, axis=1)
    h = jax.nn.relu(jnp.dot(m, w1_ref[...],
                            preferred_element_type=jnp.float32))
    logits = jnp.dot(h, w2_ref[...],
                     preferred_element_type=jnp.float32)
    lmax = jnp.max(logits, axis=1, keepdims=True)
    ex = jnp.exp(logits - lmax)
    p = ex / jnp.sum(ex, axis=1, keepdims=True)
    iota = jax.lax.broadcasted_iota(jnp.int32, p.shape, 1)
    v1 = jnp.max(p, axis=1, keepdims=True)
    i1 = jnp.min(jnp.where(p == v1, iota, E), axis=1, keepdims=True)
    p2 = jnp.where(iota == i1, -jnp.inf, p)
    v2 = jnp.max(p2, axis=1, keepdims=True)
    i2 = jnp.min(jnp.where(p2 == v2, iota, E), axis=1, keepdims=True)
    denom = v1 + v2 + 1e-6
    gates = (jnp.where(iota == i1, v1 / denom, 0.0)
             + jnp.where(iota == i2, v2 / denom, 0.0))
    gates_ref[pl.ds(i * BB, BB), :] = gates

    xr = xn.reshape(RB, L)
    grows = jnp.dot(rep_ref[...], gates,
                    preferred_element_type=jnp.float32)
    bias = bss_ref[...] + bst_ref[...]
    acc = jnp.dot(grows, bias, preferred_element_type=jnp.float32)
    for e in range(E):
        pe = jnp.dot(xr, weff_ref[e],
                     preferred_element_type=jnp.float32)
        acc = acc + grows[:, e:e + 1] * pe
    y_ref[...] = acc.reshape(BB, C, D)

    @pl.when(i == NB - 1)
    def _():
        g = gates_ref[...]
        importance = jnp.sum(g, axis=0)
        load = jnp.sum((g > 0).astype(jnp.float32), axis=0)

        def cv2(v):
            mean = jnp.mean(v)
            varr = jnp.sum((v - mean) ** 2) / (E - 1)
            return varr / (mean ** 2 + 1e-10)

        loss_ref[0, 0] = cv2(importance) + cv2(load)


def kernel(x, revin_w, revin_b, gate_W1, gate_W2, Ws_s, bs_s, Ws_t, bs_t):
    g = jnp.asarray(_G)
    rep = jnp.asarray(_REP)
    y, _, loss = pl.pallas_call(
        _body,
        grid=(NB,),
        in_specs=[
            pl.BlockSpec((BB, L, C), lambda i: (i, 0, 0)),
            pl.BlockSpec((C,), lambda i: (0,)),
            pl.BlockSpec((C,), lambda i: (0,)),
            pl.BlockSpec((L, H), lambda i: (0, 0)),
            pl.BlockSpec((H, E), lambda i: (0, 0)),
            pl.BlockSpec((E, D, L), lambda i: (0, 0, 0)),
            pl.BlockSpec((E, D), lambda i: (0, 0)),
            pl.BlockSpec((E, D, L), lambda i: (0, 0, 0)),
            pl.BlockSpec((E, D), lambda i: (0, 0)),
            pl.BlockSpec((L, 2 * L), lambda i: (0, 0)),
            pl.BlockSpec((RB, BB), lambda i: (0, 0)),
        ],
        out_specs=[
            pl.BlockSpec((BB, C, D), lambda i: (i, 0, 0)),
            pl.BlockSpec((B, E), lambda i: (0, 0)),
            pl.BlockSpec(memory_space=pltpu.SMEM),
        ],
        out_shape=[
            jax.ShapeDtypeStruct((B, C, D), jnp.float32),
            jax.ShapeDtypeStruct((B, E), jnp.float32),
            jax.ShapeDtypeStruct((1, 1), jnp.float32),
        ],
        scratch_shapes=[pltpu.VMEM((E, L, D), jnp.float32)],
    )(x, revin_w, revin_b, gate_W1, gate_W2, Ws_s, bs_s, Ws_t, bs_t, g, rep)
    return y, loss[0, 0]


# speedup vs baseline: 1.8334x; 1.4410x over previous
import jax
import jax.numpy as jnp
import numpy as np
from jax.experimental import pallas as pl
from jax.experimental.pallas import tpu as pltpu

B, L, C, D, E, H, K, KER = 512, 336, 21, 256, 8, 256, 2, 25
BB = 64
NB = B // BB
RB = BB * C


def _build_fold_matrix() -> np.ndarray:
    pad = (KER - 1) // 2
    M = np.zeros((L, L), np.float64)
    for l in range(L):
        for j in range(l - pad, l + pad + 1):
            M[l, min(max(j, 0), L - 1)] += 1.0
    M /= KER
    Mt = M.T
    return np.concatenate([np.eye(L) - Mt, Mt], axis=1).astype(np.float32)


_G = _build_fold_matrix()
_REP = np.equal.outer(np.arange(RB) // C, np.arange(BB)).astype(np.float32)


def _body(x_ref, rw_ref, rb_ref, w1_ref, w2_ref, wss_ref, bss_ref,
          wst_ref, bst_ref, g_ref, rep_ref,
          y_ref, gates_ref, loss_ref, weff_ref):
    i = pl.program_id(0)

    @pl.when(i == 0)
    def _():
        for e in range(E):
            wcat = jnp.concatenate([wss_ref[e], wst_ref[e]], axis=1)
            weff_ref[e] = jax.lax.dot_general(
                g_ref[...], wcat,
                dimension_numbers=(((1,), (1,)), ((), ())),
                preferred_element_type=jnp.float32)

    xt = x_ref[...]
    mu = jnp.mean(xt, axis=2, keepdims=True)
    var = jnp.mean(xt * xt, axis=2, keepdims=True) - mu * mu
    sd = jnp.sqrt(var + 1e-5)
    rw = rw_ref[...].reshape(1, C, 1)
    rb = rb_ref[...].reshape(1, C, 1)
    a = rw / sd
    xn = xt * a + (rb - mu * a)

    m = jnp.mean(xn, axis=1)
    h = jax.nn.relu(jnp.dot(m, w1_ref[...],
                            preferred_element_type=jnp.float32))
    logits = jnp.dot(h, w2_ref[...],
                     preferred_element_type=jnp.float32)
    lmax = jnp.max(logits, axis=1, keepdims=True)
    ex = jnp.exp(logits - lmax)
    p = ex / jnp.sum(ex, axis=1, keepdims=True)
    iota = jax.lax.broadcasted_iota(jnp.int32, p.shape, 1)
    v1 = jnp.max(p, axis=1, keepdims=True)
    i1 = jnp.min(jnp.where(p == v1, iota, E), axis=1, keepdims=True)
    p2 = jnp.where(iota == i1, -jnp.inf, p)
    v2 = jnp.max(p2, axis=1, keepdims=True)
    i2 = jnp.min(jnp.where(p2 == v2, iota, E), axis=1, keepdims=True)
    denom = v1 + v2 + 1e-6
    gates = (jnp.where(iota == i1, v1 / denom, 0.0)
             + jnp.where(iota == i2, v2 / denom, 0.0))
    gates_ref[pl.ds(i * BB, BB), :] = gates

    xr = xn.reshape(RB, L)
    grows = jnp.dot(rep_ref[...], gates,
                    preferred_element_type=jnp.float32)
    bias = bss_ref[...] + bst_ref[...]
    acc = jnp.dot(grows, bias, preferred_element_type=jnp.float32)
    for e in range(E):
        pe = jnp.dot(xr, weff_ref[e],
                     preferred_element_type=jnp.float32)
        acc = acc + grows[:, e:e + 1] * pe
    y_ref[...] = acc.reshape(BB, C, D)

    @pl.when(i == NB - 1)
    def _():
        g = gates_ref[...]
        importance = jnp.sum(g, axis=0)
        load = jnp.sum((g > 0).astype(jnp.float32), axis=0)

        def cv2(v):
            mean = jnp.mean(v)
            varr = jnp.sum((v - mean) ** 2) / (E - 1)
            return varr / (mean ** 2 + 1e-10)

        loss_ref[0, 0] = cv2(importance) + cv2(load)


def kernel(x, revin_w, revin_b, gate_W1, gate_W2, Ws_s, bs_s, Ws_t, bs_t):
    g = jnp.asarray(_G)
    rep = jnp.asarray(_REP)
    x_t = jnp.transpose(x, (0, 2, 1))
    y, _, loss = pl.pallas_call(
        _body,
        grid=(NB,),
        in_specs=[
            pl.BlockSpec((BB, C, L), lambda i: (i, 0, 0)),
            pl.BlockSpec((C,), lambda i: (0,)),
            pl.BlockSpec((C,), lambda i: (0,)),
            pl.BlockSpec((L, H), lambda i: (0, 0)),
            pl.BlockSpec((H, E), lambda i: (0, 0)),
            pl.BlockSpec((E, D, L), lambda i: (0, 0, 0)),
            pl.BlockSpec((E, D), lambda i: (0, 0)),
            pl.BlockSpec((E, D, L), lambda i: (0, 0, 0)),
            pl.BlockSpec((E, D), lambda i: (0, 0)),
            pl.BlockSpec((L, 2 * L), lambda i: (0, 0)),
            pl.BlockSpec((RB, BB), lambda i: (0, 0)),
        ],
        out_specs=[
            pl.BlockSpec((BB, C, D), lambda i: (i, 0, 0)),
            pl.BlockSpec((B, E), lambda i: (0, 0)),
            pl.BlockSpec(memory_space=pltpu.SMEM),
        ],
        out_shape=[
            jax.ShapeDtypeStruct((B, C, D), jnp.float32),
            jax.ShapeDtypeStruct((B, E), jnp.float32),
            jax.ShapeDtypeStruct((1, 1), jnp.float32),
        ],
        scratch_shapes=[pltpu.VMEM((E, L, D), jnp.float32)],
    )(x_t, revin_w, revin_b, gate_W1, gate_W2, Ws_s, bs_s, Ws_t, bs_t, g, rep)
    return y, loss[0, 0]
